# baseline (speedup 1.0000x reference)
import functools

import jax
import jax.numpy as jnp
from jax import lax
from jax.experimental import pallas as pl
from jax.experimental.pallas import tpu as pltpu
from jax.experimental.pallas import tpu_sc as plsc

N = 50000
NP = 50176
E_REAL = 850000
EP = 851968
NG = 1024
LANES = 16
NSC = 2
NSUB = 16
NTILE = NSC * NSUB
NPT = NP // NTILE
NPS = NP // NSUB

F1, F2, F3 = 96, 160, 320
CW = 32
CN1, CN2, CN3 = 3, 5, 10
PCN = F3 // LANES

EROWS = EP // 128
TROWS = 1024 * 1000 // 128
ED = 112


def _f32(*shape):
    return jax.ShapeDtypeStruct(shape, jnp.float32)


def _sc_deg(dst2d):
    mesh = plsc.VectorSubcoreMesh(core_axis_name="c", subcore_axis_name="s")
    rows_per_tile = EROWS // NTILE
    nblocks = rows_per_tile // 16

    @functools.partial(
        pl.kernel, mesh=mesh,
        compiler_params=pltpu.CompilerParams(needs_layout_passes=False, use_tc_tiling_on_sc=False),
        out_type=_f32(NTILE, NP // LANES, LANES),
        scratch_types=[
            pltpu.VMEM((16, 128), jnp.int32),
            pltpu.VMEM((NP // LANES, LANES), jnp.float32),
        ],
    )
    def k(dst_hbm, out_hbm, dblk, degt):
        wid = lax.axis_index("s") * NSC + lax.axis_index("c")
        ones = jnp.ones((LANES,), jnp.float32)

        def zero(i, _):
            degt[i, :] = jnp.zeros((LANES,), jnp.float32)
            return 0
        lax.fori_loop(0, NP // LANES, zero, 0)

        def block(b, _):
            pltpu.sync_copy(dst_hbm.at[pl.ds(wid * rows_per_tile + b * 16, 16)], dblk)

            def row(r, _):
                for jj in range(8):
                    idx = dblk[r, pl.ds(jj * LANES, LANES)]
                    plsc.addupdate_scatter(
                        degt, [idx >> 4, idx & 15], ones)
                return 0
            lax.fori_loop(0, 16, row, 0)
            return 0
        lax.fori_loop(0, nblocks, block, 0)

        pltpu.sync_copy(degt, out_hbm.at[wid])

    return k(dst2d)


def _sc_scatter(hs_rows, src2d, dst2d, cn):
    mesh = plsc.VectorSubcoreMesh(core_axis_name="c", subcore_axis_name="s")
    rows_per_tile = EROWS // NSUB
    nblocks = rows_per_tile // 16
    half_rows = rows_per_tile // 2
    odd = cn % 2 == 1
    out_types = [_f32(NP, cn * CW)]
    if odd:
        out_types.append(_f32(NP, CW))

    @functools.partial(
        pl.kernel, mesh=mesh,
        compiler_params=pltpu.CompilerParams(needs_layout_passes=False, use_tc_tiling_on_sc=False),
        out_type=out_types,
        scratch_types=[
            pltpu.VMEM((16, 128), jnp.int32),
            pltpu.VMEM((16, 128), jnp.int32),
            pltpu.VMEM((16, 128), jnp.int32),
            pltpu.VMEM((128, CW), jnp.float32),
            pltpu.VMEM((128, CW), jnp.float32),
            pltpu.VMEM((128, CW), jnp.float32),
            pltpu.VMEM((128, CW), jnp.float32),
            pltpu.VMEM((128, CW), jnp.float32),
            pltpu.VMEM_SHARED((NP, CW), jnp.float32),
            pltpu.SemaphoreType.DMA,
            pltpu.SemaphoreType.DMA,
        ],
    )
    def k(hs_hbm, src_hbm, dst_hbm, *rest):
        if odd:
            out_hbm, outx_hbm = rest[0], rest[1]
            scr = rest[2:]
        else:
            out_hbm = rest[0]
            scr = rest[1:]
        sblk, dblk, iblk, gbuf0, gbuf1, gbuf2, gbuf3, zbuf, spacc, gsem, \
            ssem = scr
        core = lax.axis_index("c")
        sub = lax.axis_index("s")

        def zf(i, _):
            zbuf[i, pl.ds(0, LANES)] = jnp.zeros((LANES,), jnp.float32)
            zbuf[i, pl.ds(LANES, LANES)] = jnp.zeros((LANES,), jnp.float32)
            return 0
        lax.fori_loop(0, 128, zf, 0)

        def zero_spacc():
            def zc(kk, _):
                pltpu.sync_copy(zbuf, spacc.at[pl.ds(sub * NPS + kk * 128, 128)])
                return 0
            lax.fori_loop(0, 24, zc, 0)
            pltpu.sync_copy(zbuf.at[pl.ds(0, 64)],
                            spacc.at[pl.ds(sub * NPS + 3072, 64)])

        def run_blocks(chunk, nblk, row_base):
            def block(b, _):
                row0 = row_base + b * 16
                pltpu.sync_copy(src_hbm.at[pl.ds(row0, 16)], sblk)
                pltpu.sync_copy(dst_hbm.at[pl.ds(row0, 16)], dblk)

                def mkidx(r, _):
                    for jj in range(8):
                        sl = pl.ds(jj * LANES, LANES)
                        iblk[r, sl] = sblk[r, sl] * cn + chunk
                    return 0
                lax.fori_loop(0, 16, mkidx, 0)

                bufs = (gbuf0, gbuf1, gbuf2, gbuf3)
                gd, sd = {}, {}
                for r in range(16):
                    if r >= 4:
                        sd[r - 4].wait()
                    gd[r] = pltpu.async_copy(
                        hs_hbm.at[iblk.at[r]], bufs[r % 4], gsem)
                    if r >= 1:
                        gd[r - 1].wait()
                        sd[r - 1] = pltpu.async_copy(
                            bufs[(r - 1) % 4],
                            spacc.at[dblk.at[r - 1]], ssem, add=True)
                gd[15].wait()
                sd[15] = pltpu.async_copy(
                    bufs[15 % 4], spacc.at[dblk.at[15]], ssem, add=True)
                for q in (12, 13, 14, 15):
                    sd[q].wait()
                return 0
            lax.fori_loop(0, nblk, block, 0)

        for p in range(cn // 2):
            chunk = 2 * p + core
            zero_spacc()
            plsc.subcore_barrier()
            run_blocks(chunk, nblocks, sub * rows_per_tile)
            plsc.subcore_barrier()
            pltpu.sync_copy(
                spacc.at[pl.ds(sub * NPS, NPS)],
                out_hbm.at[pl.ds(sub * NPS, NPS), pl.ds(chunk * CW, CW)])

        if odd:
            chunk = cn - 1
            zero_spacc()
            plsc.subcore_barrier()
            run_blocks(chunk, nblocks // 2,
                       sub * rows_per_tile + core * half_rows)
            plsc.subcore_barrier()

            @pl.when(core == 0)
            def _w0():
                pltpu.sync_copy(
                    spacc.at[pl.ds(sub * NPS, NPS)],
                    out_hbm.at[pl.ds(sub * NPS, NPS),
                               pl.ds(chunk * CW, CW)])

            @pl.when(core == 1)
            def _w1():
                pltpu.sync_copy(spacc.at[pl.ds(sub * NPS, NPS)],
                                outx_hbm.at[pl.ds(sub * NPS, NPS), :])

    res = k(hs_rows, src2d, dst2d)
    return (res[0], res[1]) if odd else (res[0], None)


def _sc_pool(acc3, dinv, batch_pad, b3p):
    mesh = plsc.VectorSubcoreMesh(core_axis_name="c", subcore_axis_name="s")

    @functools.partial(
        pl.kernel, mesh=mesh,
        compiler_params=pltpu.CompilerParams(needs_layout_passes=False, use_tc_tiling_on_sc=False),
        out_type=_f32(NTILE, NG, F3),
        scratch_types=[
            pltpu.VMEM((NPT, LANES), jnp.float32),
            pltpu.VMEM((NPT,), jnp.float32),
            pltpu.VMEM((NPT,), jnp.int32),
            pltpu.VMEM((PCN, LANES), jnp.float32),
            pltpu.VMEM((NG + 1, LANES), jnp.float32),
        ],
    )
    def k(acc_hbm, dinv_hbm, batch_hbm, b3_hbm, out_hbm, rows, dv, bt, bb, accg):
        wid = lax.axis_index("s") * NSC + lax.axis_index("c")
        n0 = wid * NPT
        pltpu.sync_copy(dinv_hbm.at[pl.ds(n0, NPT)], dv)
        pltpu.sync_copy(batch_hbm.at[pl.ds(n0, NPT)], bt)
        pltpu.sync_copy(b3_hbm, bb)
        iota = lax.iota(jnp.int32, LANES)
        neg = jnp.full((LANES,), -jnp.inf, jnp.float32)

        def chunk(c, _):
            pltpu.sync_copy(
                acc_hbm.at[pl.ds(n0, NPT), pl.ds(c * LANES, LANES)], rows)
            bv = bb[c, :]

            def ini(i, _):
                accg[i, :] = neg
                return 0
            lax.fori_loop(0, NG + 1, ini, 0)

            def group(gidx, _):
                n0 = gidx * LANES
                dvv = dv[pl.ds(n0, LANES)]
                btv = bt[pl.ds(n0, LANES)]
                for i in range(LANES):
                    val = jnp.maximum(dvv[i] * rows[n0 + i, :] + bv, 0.0)
                    gs = jnp.full((LANES,), btv[i], jnp.int32)
                    old = plsc.load_gather(accg, [gs, iota])
                    plsc.store_scatter(accg, [gs, iota], jnp.maximum(old, val))
                return 0
            lax.fori_loop(0, NPT // LANES, group, 0)

            pltpu.sync_copy(accg.at[pl.ds(0, NG)],
                            out_hbm.at[wid, :, pl.ds(c * LANES, LANES)])
            return 0
        lax.fori_loop(0, PCN, chunk, 0)

    return k(acc3, dinv, batch_pad, b3p.reshape(PCN, LANES))


def _sc_embed(table_pad, tgt2d):
    mesh = plsc.VectorSubcoreMesh(core_axis_name="c", subcore_axis_name="s")
    rows_per_tile = TROWS // NTILE
    nit = 25

    @functools.partial(
        pl.kernel, mesh=mesh,
        compiler_params=pltpu.CompilerParams(needs_layout_passes=False, use_tc_tiling_on_sc=False),
        out_type=_f32(1024 * 1000, ED),
        scratch_types=[
            pltpu.VMEM((10, 128), jnp.int32),
            pltpu.VMEM((128, ED), jnp.float32),
            pltpu.VMEM((128, ED), jnp.float32),
            pltpu.SemaphoreType.DMA,
        ],
    )
    def k(tab_hbm, tgt_hbm, out_hbm, idxb, gb0, gb1, sem):
        wid = lax.axis_index("s") * NSC + lax.axis_index("c")
        base = wid * rows_per_tile

        def it(i, _):
            pltpu.sync_copy(tgt_hbm.at[pl.ds(base + i * 10, 10)], idxb)
            bufs = (gb0, gb1)
            descs = [
                pltpu.async_copy(tab_hbm.at[idxb.at[0]], gb0, sem),
                pltpu.async_copy(tab_hbm.at[idxb.at[1]], gb1, sem),
            ]
            for r in range(10):
                descs[r].wait()
                pltpu.sync_copy(
                    bufs[r % 2],
                    out_hbm.at[pl.ds((base + i * 10 + r) * 128, 128)])
                if r + 2 < 10:
                    descs.append(pltpu.async_copy(
                        tab_hbm.at[idxb.at[r + 2]], bufs[r % 2], sem))
            return 0
        lax.fori_loop(0, nit, it, 0)

    return k(table_pad, tgt2d)


RB = 6272


def _tc_m1(x_pad, w1p, degp):
    def body(x_ref, w_ref, deg_ref, hs_ref, dinv_ref):
        deg = jnp.sum(deg_ref[...], axis=0)
        dinv = lax.rsqrt(jnp.maximum(deg, 0.5))
        h = jnp.dot(x_ref[...], w_ref[...], preferred_element_type=jnp.float32)
        hs_ref[...] = h * dinv[:, None]
        dinv_ref[...] = dinv[:, None]

    return pl.pallas_call(
        body,
        grid=(NP // RB,),
        in_specs=[
            pl.BlockSpec((RB, F1), lambda i: (i, 0)),
            pl.BlockSpec((F1, F1), lambda i: (0, 0)),
            pl.BlockSpec((NTILE, RB), lambda i: (0, i)),
        ],
        out_specs=[
            pl.BlockSpec((RB, F1), lambda i: (i, 0)),
            pl.BlockSpec((RB, 1), lambda i: (i, 0)),
        ],
        out_shape=[_f32(NP, F1), _f32(NP, 1)],
    )(x_pad, w1p, degp)


def _tc_m23(acc, accx, dinv, bprev, w, fin, fout):
    def body(acc_ref, accx_ref, dinv_ref, b_ref, w_ref, out_ref):
        dv = dinv_ref[...]
        full = jnp.concatenate(
            [acc_ref[:, : fin - CW], acc_ref[:, fin - CW:] + accx_ref[...]],
            axis=1)
        a = jnp.maximum(full * dv + b_ref[...][None, :], 0.0)
        h = jnp.dot(a, w_ref[...], preferred_element_type=jnp.float32)
        out_ref[...] = h * dv

    return pl.pallas_call(
        body,
        grid=(NP // RB,),
        in_specs=[
            pl.BlockSpec((RB, fin), lambda i: (i, 0)),
            pl.BlockSpec((RB, CW), lambda i: (i, 0)),
            pl.BlockSpec((RB, 1), lambda i: (i, 0)),
            pl.BlockSpec((fin,), lambda i: (0,)),
            pl.BlockSpec((fin, fout), lambda i: (0, 0)),
        ],
        out_specs=pl.BlockSpec((RB, fout), lambda i: (i, 0)),
        out_shape=_f32(NP, fout),
    )(acc, accx, dinv, bprev, w)


def _tc_pmm(emb2, u_flat):
    kb = 3200
    nk = 112000 // kb

    def body(e_ref, u_ref, out_ref):
        @pl.when(pl.program_id(0) == 0)
        def _():
            out_ref[...] = jnp.zeros_like(out_ref)
        out_ref[...] += jnp.dot(e_ref[...], u_ref[...],
                                precision=lax.Precision.HIGHEST,
                                preferred_element_type=jnp.float32)

    return pl.pallas_call(
        body,
        grid=(nk,),
        in_specs=[
            pl.BlockSpec((NG, kb), lambda k: (0, k)),
            pl.BlockSpec((kb, 128), lambda k: (k, 0)),
        ],
        out_specs=pl.BlockSpec((NG, 128), lambda k: (0, 0)),
        out_shape=_f32(NG, 128),
    )(emb2, u_flat)


def _tc_head(partials, p_raw, wg1p, bg1, wg2, bg2, bxt, wm1, bm1, wm2, bm2,
             wm3, bm3):
    bb = 128

    def body(part_ref, p_ref, wg1_ref, bg1_ref, wg2_ref, bg2_ref, bxt_ref,
             wm1_ref, bm1_ref, wm2_ref, bm2_ref, wm3_ref, bm3_ref, out_ref):
        hp = lax.Precision.HIGHEST
        pool = jnp.max(part_ref[...], axis=0)
        g = jnp.maximum(jnp.dot(pool, wg1_ref[...], precision=hp,
                                preferred_element_type=jnp.float32)
                        + bg1_ref[...][None, :], 0.0)
        g = jnp.dot(g, wg2_ref[...], precision=hp,
                    preferred_element_type=jnp.float32) \
            + bg2_ref[...][None, :]
        p = p_ref[...] + bxt_ref[...][None, :]
        hid = jnp.concatenate([g, p], axis=1)
        hid = jnp.maximum(jnp.dot(hid, wm1_ref[...], precision=hp,
                                  preferred_element_type=jnp.float32)
                          + bm1_ref[...][None, :], 0.0)
        hid = jnp.maximum(jnp.dot(hid, wm2_ref[...], precision=hp,
                                  preferred_element_type=jnp.float32)
                          + bm2_ref[...][None, :], 0.0)
        out_ref[...] = jnp.dot(hid, wm3_ref[...], precision=hp,
                               preferred_element_type=jnp.float32) \
            + bm3_ref[...][None, :]

    return pl.pallas_call(
        body,
        grid=(NG // bb,),
        in_specs=[
            pl.BlockSpec((NTILE, bb, F3), lambda i: (0, i, 0)),
            pl.BlockSpec((bb, 128), lambda i: (i, 0)),
            pl.BlockSpec((F3, 1024), lambda i: (0, 0)),
            pl.BlockSpec((1024,), lambda i: (0,)),
            pl.BlockSpec((1024, 128), lambda i: (0, 0)),
            pl.BlockSpec((128,), lambda i: (0,)),
            pl.BlockSpec((128,), lambda i: (0,)),
            pl.BlockSpec((256, 1024), lambda i: (0, 0)),
            pl.BlockSpec((1024,), lambda i: (0,)),
            pl.BlockSpec((1024, 512), lambda i: (0, 0)),
            pl.BlockSpec((512,), lambda i: (0,)),
            pl.BlockSpec((512, 1), lambda i: (0, 0)),
            pl.BlockSpec((1,), lambda i: (0,)),
        ],
        out_specs=pl.BlockSpec((bb, 1), lambda i: (i, 0)),
        out_shape=_f32(NG, 1),
    )(partials, p_raw, wg1p, bg1, wg2, bg2, bxt, wm1, bm1, wm2, bm2, wm3, bm3)


def kernel(x, edge_index, batch, target, W1, b1, W2, b2, W3, b3, Wg1, bg1,
           Wg2, bg2, emb_table, Wc, bc, Wxt, bxt, Wm1, bm1, Wm2, bm2, Wm3,
           bm3):
    f32 = jnp.float32
    x_pad = jnp.zeros((NP, F1), f32).at[:N, :78].set(x)
    w1p = jnp.zeros((F1, F1), f32).at[:78, :78].set(W1)
    w2p = jnp.zeros((F1, F2), f32).at[:78, :156].set(W2)
    w3p = jnp.zeros((F2, F3), f32).at[:156, :312].set(W3)
    b1p = jnp.zeros((F1,), f32).at[:78].set(b1)
    b2p = jnp.zeros((F2,), f32).at[:156].set(b2)
    b3p = jnp.zeros((F3,), f32).at[:312].set(b3)
    wg1p = jnp.zeros((F3, 1024), f32).at[:312, :].set(Wg1)

    loop = jnp.arange(N, dtype=jnp.int32)
    padv = jnp.full((EP - E_REAL,), N, jnp.int32)
    src = jnp.concatenate([edge_index[0], loop, padv]).reshape(EROWS, 128)
    dst = jnp.concatenate([edge_index[1], loop, padv]).reshape(EROWS, 128)
    batch_pad = jnp.concatenate(
        [batch, jnp.full((NP - N,), NG, jnp.int32)])

    table_pad = jnp.zeros((8000, ED), f32).at[:, :100].set(emb_table)
    tgt2d = target.reshape(TROWS, 128)

    wxt3 = Wxt.reshape(32, 93, 128)
    u = jnp.zeros((1000, ED, 128), f32)
    for kk in range(8):
        t_k = jnp.einsum('oc,opm->cpm', Wc[:, :, kk], wxt3,
                         precision=jax.lax.Precision.HIGHEST)
        u = u.at[:, kk:kk + 93, :].add(t_k)
    bxt_eff = bxt + jnp.einsum('o,opm->m', bc, wxt3,
                               precision=jax.lax.Precision.HIGHEST)
    u_flat = u.reshape(112000, 128)

    emb = _sc_embed(table_pad, tgt2d)
    p_raw = _tc_pmm(emb.reshape(NG, 112000), u_flat)
    tok = (emb[0, 0] * 0.0).astype(jnp.int32)

    degp = _sc_deg(dst + tok).reshape(NTILE, NP)
    hs1, dinv = _tc_m1(x_pad, w1p, degp)
    acc1, acc1x = _sc_scatter(hs1.reshape(NP * CN1, CW), src, dst, CN1)
    hs2 = _tc_m23(acc1, acc1x, dinv, b1p, w2p, F1, F2)
    acc2, acc2x = _sc_scatter(hs2.reshape(NP * CN2, CW), src, dst, CN2)
    hs3 = _tc_m23(acc2, acc2x, dinv, b2p, w3p, F2, F3)
    acc3, _unused = _sc_scatter(hs3.reshape(NP * CN3, CW), src, dst, CN3)
    partials = _sc_pool(acc3, dinv.reshape(NP), batch_pad, b3p)

    return _tc_head(partials, p_raw, wg1p, bg1, Wg2, bg2, bxt_eff, Wm1, bm1,
                    Wm2, bm2, Wm3, bm3)


# speedup vs baseline: 1.1681x; 1.1681x over previous
import functools

import jax
import jax.numpy as jnp
from jax import lax
from jax.experimental import pallas as pl
from jax.experimental.pallas import tpu as pltpu
from jax.experimental.pallas import tpu_sc as plsc

N = 50000
NP = 50176
E_REAL = 850000
EP = 851968
NG = 1024
LANES = 16
NSC = 2
NSUB = 16
NTILE = NSC * NSUB
NPT = NP // NTILE
NPS = NP // NSUB

F1, F2, F3 = 96, 160, 320
CW = 32
CN1, CN2, CN3 = 3, 5, 10
PCN = F3 // LANES

EROWS = EP // 128
TROWS = 1024 * 1000 // 128
ED = 112


def _f32(*shape):
    return jax.ShapeDtypeStruct(shape, jnp.float32)


def _sc_deg(dst2d):
    mesh = plsc.VectorSubcoreMesh(core_axis_name="c", subcore_axis_name="s")
    rows_per_tile = EROWS // NTILE
    nblocks = rows_per_tile // 16

    @functools.partial(
        pl.kernel, mesh=mesh,
        compiler_params=pltpu.CompilerParams(needs_layout_passes=False, use_tc_tiling_on_sc=False),
        out_type=_f32(NTILE, NP // LANES, LANES),
        scratch_types=[
            pltpu.VMEM((16, 128), jnp.int32),
            pltpu.VMEM((NP // LANES, LANES), jnp.float32),
        ],
    )
    def k(dst_hbm, out_hbm, dblk, degt):
        wid = lax.axis_index("s") * NSC + lax.axis_index("c")
        ones = jnp.ones((LANES,), jnp.float32)

        def zero(i, _):
            degt[i, :] = jnp.zeros((LANES,), jnp.float32)
            return 0
        lax.fori_loop(0, NP // LANES, zero, 0)

        def block(b, _):
            pltpu.sync_copy(dst_hbm.at[pl.ds(wid * rows_per_tile + b * 16, 16)], dblk)

            def row(r, _):
                for jj in range(8):
                    idx = dblk[r, pl.ds(jj * LANES, LANES)]
                    plsc.addupdate_scatter(
                        degt, [idx >> 4, idx & 15], ones)
                return 0
            lax.fori_loop(0, 16, row, 0)
            return 0
        lax.fori_loop(0, nblocks, block, 0)

        pltpu.sync_copy(degt, out_hbm.at[wid])

    return k(dst2d)


def _sc_scatter(hs_rows, src2d, dst2d, cn):
    mesh = plsc.VectorSubcoreMesh(core_axis_name="c", subcore_axis_name="s")
    rows_per_tile = EROWS // NSUB
    nblocks = rows_per_tile // 16
    half_rows = rows_per_tile // 2
    odd = cn % 2 == 1
    out_types = [_f32(NP, cn * CW)]
    if odd:
        out_types.append(_f32(NP, CW))

    @functools.partial(
        pl.kernel, mesh=mesh,
        compiler_params=pltpu.CompilerParams(needs_layout_passes=False, use_tc_tiling_on_sc=False),
        out_type=out_types,
        scratch_types=[
            pltpu.VMEM((16, 128), jnp.int32),
            pltpu.VMEM((16, 128), jnp.int32),
            pltpu.VMEM((16, 128), jnp.int32),
            pltpu.VMEM((128, CW), jnp.float32),
            pltpu.VMEM((128, CW), jnp.float32),
            pltpu.VMEM((128, CW), jnp.float32),
            pltpu.VMEM((128, CW), jnp.float32),
            pltpu.VMEM((128, CW), jnp.float32),
            pltpu.VMEM_SHARED((NP, CW), jnp.float32),
            pltpu.SemaphoreType.DMA,
            pltpu.SemaphoreType.DMA,
        ],
    )
    def k(hs_hbm, src_hbm, dst_hbm, *rest):
        if odd:
            out_hbm, outx_hbm = rest[0], rest[1]
            scr = rest[2:]
        else:
            out_hbm = rest[0]
            scr = rest[1:]
        sblk, dblk, iblk, gbuf0, gbuf1, gbuf2, gbuf3, zbuf, spacc, gsem, \
            ssem = scr
        core = lax.axis_index("c")
        sub = lax.axis_index("s")

        def zf(i, _):
            zbuf[i, pl.ds(0, LANES)] = jnp.zeros((LANES,), jnp.float32)
            zbuf[i, pl.ds(LANES, LANES)] = jnp.zeros((LANES,), jnp.float32)
            return 0
        lax.fori_loop(0, 128, zf, 0)

        def zero_spacc():
            def zc(kk, _):
                pltpu.sync_copy(zbuf, spacc.at[pl.ds(sub * NPS + kk * 128, 128)])
                return 0
            lax.fori_loop(0, 24, zc, 0)
            pltpu.sync_copy(zbuf.at[pl.ds(0, 64)],
                            spacc.at[pl.ds(sub * NPS + 3072, 64)])

        def run_blocks(chunk, nblk, row_base):
            def block(b, _):
                row0 = row_base + b * 16
                pltpu.sync_copy(src_hbm.at[pl.ds(row0, 16)], sblk)
                pltpu.sync_copy(dst_hbm.at[pl.ds(row0, 16)], dblk)

                def mkidx(r, _):
                    for jj in range(8):
                        sl = pl.ds(jj * LANES, LANES)
                        iblk[r, sl] = sblk[r, sl] * cn + chunk
                    return 0
                lax.fori_loop(0, 16, mkidx, 0)

                bufs = (gbuf0, gbuf1, gbuf2, gbuf3)
                gd, sd = {}, {}
                for r in range(16):
                    if r >= 4:
                        sd[r - 4].wait()
                    gd[r] = pltpu.async_copy(
                        hs_hbm.at[iblk.at[r]], bufs[r % 4], gsem)
                    if r >= 1:
                        gd[r - 1].wait()
                        sd[r - 1] = pltpu.async_copy(
                            bufs[(r - 1) % 4],
                            spacc.at[dblk.at[r - 1]], ssem, add=True)
                gd[15].wait()
                sd[15] = pltpu.async_copy(
                    bufs[15 % 4], spacc.at[dblk.at[15]], ssem, add=True)
                for q in (12, 13, 14, 15):
                    sd[q].wait()
                return 0
            lax.fori_loop(0, nblk, block, 0)

        for p in range(cn // 2):
            chunk = 2 * p + core
            zero_spacc()
            plsc.subcore_barrier()
            run_blocks(chunk, nblocks, sub * rows_per_tile)
            plsc.subcore_barrier()
            pltpu.sync_copy(
                spacc.at[pl.ds(sub * NPS, NPS)],
                out_hbm.at[pl.ds(sub * NPS, NPS), pl.ds(chunk * CW, CW)])

        if odd:
            chunk = cn - 1
            zero_spacc()
            plsc.subcore_barrier()
            run_blocks(chunk, nblocks // 2,
                       sub * rows_per_tile + core * half_rows)
            plsc.subcore_barrier()

            @pl.when(core == 0)
            def _w0():
                pltpu.sync_copy(
                    spacc.at[pl.ds(sub * NPS, NPS)],
                    out_hbm.at[pl.ds(sub * NPS, NPS),
                               pl.ds(chunk * CW, CW)])

            @pl.when(core == 1)
            def _w1():
                pltpu.sync_copy(spacc.at[pl.ds(sub * NPS, NPS)],
                                outx_hbm.at[pl.ds(sub * NPS, NPS), :])

    res = k(hs_rows, src2d, dst2d)
    return (res[0], res[1]) if odd else (res[0], None)


def _sc_pool(acc3, dinv, batch_pad, b3p):
    mesh = plsc.VectorSubcoreMesh(core_axis_name="c", subcore_axis_name="s")

    @functools.partial(
        pl.kernel, mesh=mesh,
        compiler_params=pltpu.CompilerParams(needs_layout_passes=False, use_tc_tiling_on_sc=False),
        out_type=_f32(NTILE, NG, F3),
        scratch_types=[
            pltpu.VMEM((NPT, 2 * LANES), jnp.float32),
            pltpu.VMEM((NPT,), jnp.float32),
            pltpu.VMEM((NPT,), jnp.int32),
            pltpu.VMEM((PCN, LANES), jnp.float32),
            pltpu.VMEM((NG + 1, 2 * LANES), jnp.float32),
        ],
    )
    def k(acc_hbm, dinv_hbm, batch_hbm, b3_hbm, out_hbm, rows, dv, bt, bb, accg):
        wid = lax.axis_index("s") * NSC + lax.axis_index("c")
        n0 = wid * NPT
        pltpu.sync_copy(dinv_hbm.at[pl.ds(n0, NPT)], dv)
        pltpu.sync_copy(batch_hbm.at[pl.ds(n0, NPT)], bt)
        pltpu.sync_copy(b3_hbm, bb)
        iota = lax.iota(jnp.int32, LANES)
        neg = jnp.full((LANES,), -jnp.inf, jnp.float32)

        iota2 = iota + LANES

        def chunk(cp, _):
            pltpu.sync_copy(
                acc_hbm.at[pl.ds(n0, NPT), pl.ds(cp * 2 * LANES, 2 * LANES)],
                rows)
            bv0 = bb[2 * cp, :]
            bv1 = bb[2 * cp + 1, :]

            def ini(i, _):
                accg[i, pl.ds(0, LANES)] = neg
                accg[i, pl.ds(LANES, LANES)] = neg
                return 0
            lax.fori_loop(0, NG + 1, ini, 0)

            def group(gidx, _):
                nb = gidx * LANES
                dvv = dv[pl.ds(nb, LANES)]
                btv = bt[pl.ds(nb, LANES)]
                for i in range(LANES):
                    r0 = rows[nb + i, pl.ds(0, LANES)]
                    r1 = rows[nb + i, pl.ds(LANES, LANES)]
                    val0 = jnp.maximum(dvv[i] * r0 + bv0, 0.0)
                    val1 = jnp.maximum(dvv[i] * r1 + bv1, 0.0)
                    gs = jnp.full((LANES,), btv[i], jnp.int32)
                    old0 = plsc.load_gather(accg, [gs, iota])
                    old1 = plsc.load_gather(accg, [gs, iota2])
                    plsc.store_scatter(accg, [gs, iota],
                                       jnp.maximum(old0, val0))
                    plsc.store_scatter(accg, [gs, iota2],
                                       jnp.maximum(old1, val1))
                return 0
            lax.fori_loop(0, NPT // LANES, group, 0)

            pltpu.sync_copy(
                accg.at[pl.ds(0, NG)],
                out_hbm.at[wid, :, pl.ds(cp * 2 * LANES, 2 * LANES)])
            return 0
        lax.fori_loop(0, PCN // 2, chunk, 0)

    return k(acc3, dinv, batch_pad, b3p.reshape(PCN, LANES))


def _sc_embed(table_pad, tgt2d):
    mesh = plsc.VectorSubcoreMesh(core_axis_name="c", subcore_axis_name="s")
    rows_per_tile = TROWS // NTILE
    nit = 25

    @functools.partial(
        pl.kernel, mesh=mesh,
        compiler_params=pltpu.CompilerParams(needs_layout_passes=False, use_tc_tiling_on_sc=False),
        out_type=_f32(1024 * 1000, ED),
        scratch_types=[
            pltpu.VMEM((10, 128), jnp.int32),
            pltpu.VMEM((128, ED), jnp.float32),
            pltpu.VMEM((128, ED), jnp.float32),
            pltpu.SemaphoreType.DMA,
        ],
    )
    def k(tab_hbm, tgt_hbm, out_hbm, idxb, gb0, gb1, sem):
        wid = lax.axis_index("s") * NSC + lax.axis_index("c")
        base = wid * rows_per_tile

        def it(i, _):
            pltpu.sync_copy(tgt_hbm.at[pl.ds(base + i * 10, 10)], idxb)
            bufs = (gb0, gb1)
            descs = [
                pltpu.async_copy(tab_hbm.at[idxb.at[0]], gb0, sem),
                pltpu.async_copy(tab_hbm.at[idxb.at[1]], gb1, sem),
            ]
            for r in range(10):
                descs[r].wait()
                pltpu.sync_copy(
                    bufs[r % 2],
                    out_hbm.at[pl.ds((base + i * 10 + r) * 128, 128)])
                if r + 2 < 10:
                    descs.append(pltpu.async_copy(
                        tab_hbm.at[idxb.at[r + 2]], bufs[r % 2], sem))
            return 0
        lax.fori_loop(0, nit, it, 0)

    return k(table_pad, tgt2d)


RB = 6272


def _tc_m1(x_pad, w1p, degp):
    def body(x_ref, w_ref, deg_ref, hs_ref, dinv_ref):
        deg = jnp.sum(deg_ref[...], axis=0)
        dinv = lax.rsqrt(jnp.maximum(deg, 0.5))
        h = jnp.dot(x_ref[...], w_ref[...], preferred_element_type=jnp.float32)
        hs_ref[...] = h * dinv[:, None]
        dinv_ref[...] = dinv[:, None]

    return pl.pallas_call(
        body,
        grid=(NP // RB,),
        in_specs=[
            pl.BlockSpec((RB, F1), lambda i: (i, 0)),
            pl.BlockSpec((F1, F1), lambda i: (0, 0)),
            pl.BlockSpec((NTILE, RB), lambda i: (0, i)),
        ],
        out_specs=[
            pl.BlockSpec((RB, F1), lambda i: (i, 0)),
            pl.BlockSpec((RB, 1), lambda i: (i, 0)),
        ],
        out_shape=[_f32(NP, F1), _f32(NP, 1)],
    )(x_pad, w1p, degp)


def _tc_m23(acc, accx, dinv, bprev, w, fin, fout):
    def body(acc_ref, accx_ref, dinv_ref, b_ref, w_ref, out_ref):
        dv = dinv_ref[...]
        full = jnp.concatenate(
            [acc_ref[:, : fin - CW], acc_ref[:, fin - CW:] + accx_ref[...]],
            axis=1)
        a = jnp.maximum(full * dv + b_ref[...][None, :], 0.0)
        h = jnp.dot(a, w_ref[...], preferred_element_type=jnp.float32)
        out_ref[...] = h * dv

    return pl.pallas_call(
        body,
        grid=(NP // RB,),
        in_specs=[
            pl.BlockSpec((RB, fin), lambda i: (i, 0)),
            pl.BlockSpec((RB, CW), lambda i: (i, 0)),
            pl.BlockSpec((RB, 1), lambda i: (i, 0)),
            pl.BlockSpec((fin,), lambda i: (0,)),
            pl.BlockSpec((fin, fout), lambda i: (0, 0)),
        ],
        out_specs=pl.BlockSpec((RB, fout), lambda i: (i, 0)),
        out_shape=_f32(NP, fout),
    )(acc, accx, dinv, bprev, w)


def _tc_pmm(emb2, u_flat):
    kb = 3200
    nk = 112000 // kb

    def body(e_ref, u_ref, out_ref):
        @pl.when(pl.program_id(0) == 0)
        def _():
            out_ref[...] = jnp.zeros_like(out_ref)
        out_ref[...] += jnp.dot(e_ref[...], u_ref[...],
                                preferred_element_type=jnp.float32)

    return pl.pallas_call(
        body,
        grid=(nk,),
        in_specs=[
            pl.BlockSpec((NG, kb), lambda k: (0, k)),
            pl.BlockSpec((kb, 128), lambda k: (k, 0)),
        ],
        out_specs=pl.BlockSpec((NG, 128), lambda k: (0, 0)),
        out_shape=_f32(NG, 128),
    )(emb2, u_flat)


def _tc_head(partials, p_raw, wg1p, bg1, wg2, bg2, bxt, wm1, bm1, wm2, bm2,
             wm3, bm3):
    bb = 128

    def body(part_ref, p_ref, wg1_ref, bg1_ref, wg2_ref, bg2_ref, bxt_ref,
             wm1_ref, bm1_ref, wm2_ref, bm2_ref, wm3_ref, bm3_ref, out_ref):
        hp = lax.Precision.HIGHEST
        pool = jnp.max(part_ref[...], axis=0)
        g = jnp.maximum(jnp.dot(pool, wg1_ref[...], precision=hp,
                                preferred_element_type=jnp.float32)
                        + bg1_ref[...][None, :], 0.0)
        g = jnp.dot(g, wg2_ref[...], precision=hp,
                    preferred_element_type=jnp.float32) \
            + bg2_ref[...][None, :]
        p = p_ref[...] + bxt_ref[...][None, :]
        hid = jnp.concatenate([g, p], axis=1)
        hid = jnp.maximum(jnp.dot(hid, wm1_ref[...], precision=hp,
                                  preferred_element_type=jnp.float32)
                          + bm1_ref[...][None, :], 0.0)
        hid = jnp.maximum(jnp.dot(hid, wm2_ref[...], precision=hp,
                                  preferred_element_type=jnp.float32)
                          + bm2_ref[...][None, :], 0.0)
        out_ref[...] = jnp.dot(hid, wm3_ref[...], precision=hp,
                               preferred_element_type=jnp.float32) \
            + bm3_ref[...][None, :]

    return pl.pallas_call(
        body,
        grid=(NG // bb,),
        in_specs=[
            pl.BlockSpec((NTILE, bb, F3), lambda i: (0, i, 0)),
            pl.BlockSpec((bb, 128), lambda i: (i, 0)),
            pl.BlockSpec((F3, 1024), lambda i: (0, 0)),
            pl.BlockSpec((1024,), lambda i: (0,)),
            pl.BlockSpec((1024, 128), lambda i: (0, 0)),
            pl.BlockSpec((128,), lambda i: (0,)),
            pl.BlockSpec((128,), lambda i: (0,)),
            pl.BlockSpec((256, 1024), lambda i: (0, 0)),
            pl.BlockSpec((1024,), lambda i: (0,)),
            pl.BlockSpec((1024, 512), lambda i: (0, 0)),
            pl.BlockSpec((512,), lambda i: (0,)),
            pl.BlockSpec((512, 1), lambda i: (0, 0)),
            pl.BlockSpec((1,), lambda i: (0,)),
        ],
        out_specs=pl.BlockSpec((bb, 1), lambda i: (i, 0)),
        out_shape=_f32(NG, 1),
    )(partials, p_raw, wg1p, bg1, wg2, bg2, bxt, wm1, bm1, wm2, bm2, wm3, bm3)


def kernel(x, edge_index, batch, target, W1, b1, W2, b2, W3, b3, Wg1, bg1,
           Wg2, bg2, emb_table, Wc, bc, Wxt, bxt, Wm1, bm1, Wm2, bm2, Wm3,
           bm3):
    f32 = jnp.float32
    x_pad = jnp.zeros((NP, F1), f32).at[:N, :78].set(x)
    w1p = jnp.zeros((F1, F1), f32).at[:78, :78].set(W1)
    w2p = jnp.zeros((F1, F2), f32).at[:78, :156].set(W2)
    w3p = jnp.zeros((F2, F3), f32).at[:156, :312].set(W3)
    b1p = jnp.zeros((F1,), f32).at[:78].set(b1)
    b2p = jnp.zeros((F2,), f32).at[:156].set(b2)
    b3p = jnp.zeros((F3,), f32).at[:312].set(b3)
    wg1p = jnp.zeros((F3, 1024), f32).at[:312, :].set(Wg1)

    loop = jnp.arange(N, dtype=jnp.int32)
    padv = jnp.full((EP - E_REAL,), N, jnp.int32)
    src = jnp.concatenate([edge_index[0], loop, padv]).reshape(EROWS, 128)
    dst = jnp.concatenate([edge_index[1], loop, padv]).reshape(EROWS, 128)
    batch_pad = jnp.concatenate(
        [batch, jnp.full((NP - N,), NG, jnp.int32)])

    table_pad = jnp.zeros((8000, ED), f32).at[:, :100].set(emb_table)
    tgt2d = target.reshape(TROWS, 128)

    wxt3 = Wxt.reshape(32, 93, 128)
    u = jnp.zeros((1000, ED, 128), f32)
    for kk in range(8):
        t_k = jnp.einsum('oc,opm->cpm', Wc[:, :, kk], wxt3,
                         precision=jax.lax.Precision.HIGHEST)
        u = u.at[:, kk:kk + 93, :].add(t_k)
    bxt_eff = bxt + jnp.einsum('o,opm->m', bc, wxt3,
                               precision=jax.lax.Precision.HIGHEST)
    u_flat = u.reshape(112000, 128)

    emb = _sc_embed(table_pad, tgt2d)
    p_raw = _tc_pmm(emb.reshape(NG, 112000), u_flat)

    degp = _sc_deg(dst).reshape(NTILE, NP)
    hs1, dinv = _tc_m1(x_pad, w1p, degp)
    acc1, acc1x = _sc_scatter(hs1.reshape(NP * CN1, CW), src, dst, CN1)
    hs2 = _tc_m23(acc1, acc1x, dinv, b1p, w2p, F1, F2)
    acc2, acc2x = _sc_scatter(hs2.reshape(NP * CN2, CW), src, dst, CN2)
    hs3 = _tc_m23(acc2, acc2x, dinv, b2p, w3p, F2, F3)
    acc3, _unused = _sc_scatter(hs3.reshape(NP * CN3, CW), src, dst, CN3)
    partials = _sc_pool(acc3, dinv.reshape(NP), batch_pad, b3p)

    return _tc_head(partials, p_raw, wg1p, bg1, Wg2, bg2, bxt_eff, Wm1, bm1,
                    Wm2, bm2, Wm3, bm3)


# speedup vs baseline: 1.2399x; 1.0614x over previous
import functools

import jax
import jax.numpy as jnp
from jax import lax
from jax.experimental import pallas as pl
from jax.experimental.pallas import tpu as pltpu
from jax.experimental.pallas import tpu_sc as plsc

N = 50000
NP = 50176
E_REAL = 850000
EP = 851968
NG = 1024
LANES = 16
NSC = 2
NSUB = 16
NTILE = NSC * NSUB
NPT = NP // NTILE
NPS = NP // NSUB

F1, F2, F3 = 96, 160, 320
CW = 32
CN1, CN2, CN3 = 3, 5, 10
PCN = F3 // LANES

EROWS = EP // 128
TROWS = 1024 * 1000 // 128
ED = 112


def _f32(*shape):
    return jax.ShapeDtypeStruct(shape, jnp.float32)


def _sc_deg(dst2d):
    mesh = plsc.VectorSubcoreMesh(core_axis_name="c", subcore_axis_name="s")
    rows_per_tile = EROWS // NTILE
    nblocks = rows_per_tile // 16

    @functools.partial(
        pl.kernel, mesh=mesh,
        compiler_params=pltpu.CompilerParams(needs_layout_passes=False, use_tc_tiling_on_sc=False),
        out_type=_f32(NTILE, NP // LANES, LANES),
        scratch_types=[
            pltpu.VMEM((16, 128), jnp.int32),
            pltpu.VMEM((NP // LANES, LANES), jnp.float32),
        ],
    )
    def k(dst_hbm, out_hbm, dblk, degt):
        wid = lax.axis_index("s") * NSC + lax.axis_index("c")
        ones = jnp.ones((LANES,), jnp.float32)

        def zero(i, _):
            degt[i, :] = jnp.zeros((LANES,), jnp.float32)
            return 0
        lax.fori_loop(0, NP // LANES, zero, 0)

        def block(b, _):
            pltpu.sync_copy(dst_hbm.at[pl.ds(wid * rows_per_tile + b * 16, 16)], dblk)

            def row(r, _):
                for jj in range(8):
                    idx = dblk[r, pl.ds(jj * LANES, LANES)]
                    plsc.addupdate_scatter(
                        degt, [idx >> 4, idx & 15], ones)
                return 0
            lax.fori_loop(0, 16, row, 0)
            return 0
        lax.fori_loop(0, nblocks, block, 0)

        pltpu.sync_copy(degt, out_hbm.at[wid])

    return k(dst2d)


def _sc_scatter(hs_rows, src2d, dst2d, cn):
    mesh = plsc.VectorSubcoreMesh(core_axis_name="c", subcore_axis_name="s")
    rows_per_tile = EROWS // NSUB
    nblocks = rows_per_tile // 16
    half_rows = rows_per_tile // 2
    odd = cn % 2 == 1
    out_types = [_f32(NP, cn * CW)]
    if odd:
        out_types.append(_f32(NP, CW))

    @functools.partial(
        pl.kernel, mesh=mesh,
        compiler_params=pltpu.CompilerParams(needs_layout_passes=False, use_tc_tiling_on_sc=False),
        out_type=out_types,
        scratch_types=[
            pltpu.VMEM((16, 128), jnp.int32),
            pltpu.VMEM((16, 128), jnp.int32),
            pltpu.VMEM((16, 128), jnp.int32),
            pltpu.VMEM((16, 128), jnp.int32),
            pltpu.VMEM((128, CW), jnp.float32),
            pltpu.VMEM((128, CW), jnp.float32),
            pltpu.VMEM((128, CW), jnp.float32),
            pltpu.VMEM((128, CW), jnp.float32),
            pltpu.VMEM((128, CW), jnp.float32),
            pltpu.VMEM_SHARED((NP, CW), jnp.float32),
            pltpu.SemaphoreType.DMA,
            pltpu.SemaphoreType.DMA,
            pltpu.SemaphoreType.DMA,
        ],
    )
    def k(hs_hbm, src_hbm, dst_hbm, *rest):
        if odd:
            out_hbm, outx_hbm = rest[0], rest[1]
            scr = rest[2:]
        else:
            out_hbm = rest[0]
            scr = rest[1:]
        sblkA, dblkA, sblkB, dblkB, gbuf0, gbuf1, gbuf2, gbuf3, zbuf, \
            spacc, gsem, ssem, isem = scr
        core = lax.axis_index("c")
        sub = lax.axis_index("s")

        def zf(i, _):
            zbuf[i, pl.ds(0, LANES)] = jnp.zeros((LANES,), jnp.float32)
            zbuf[i, pl.ds(LANES, LANES)] = jnp.zeros((LANES,), jnp.float32)
            return 0
        lax.fori_loop(0, 128, zf, 0)

        def zero_spacc():
            def zc(kk, _):
                pltpu.sync_copy(zbuf, spacc.at[pl.ds(sub * NPS + kk * 128, 128)])
                return 0
            lax.fori_loop(0, 24, zc, 0)
            pltpu.sync_copy(zbuf.at[pl.ds(0, 64)],
                            spacc.at[pl.ds(sub * NPS + 3072, 64)])

        def run_blocks(chunk, nblk, row_base):
            def stage(b, sb, db):
                row0 = row_base + b * 16
                pltpu.async_copy(src_hbm.at[pl.ds(row0, 16)], sb, isem)
                pltpu.async_copy(dst_hbm.at[pl.ds(row0, 16)], db, isem)

            def drain(sb, db):
                pltpu.make_async_copy(src_hbm.at[pl.ds(0, 16)], sb,
                                      isem).wait()
                pltpu.make_async_copy(dst_hbm.at[pl.ds(0, 16)], db,
                                      isem).wait()

            def one_block(sblk, dblk):
                def mkidx(r, _):
                    for jj in range(8):
                        sl = pl.ds(jj * LANES, LANES)
                        sblk[r, sl] = sblk[r, sl] * cn + chunk
                    return 0
                lax.fori_loop(0, 16, mkidx, 0)

                bufs = (gbuf0, gbuf1, gbuf2, gbuf3)
                gd, sd = {}, {}
                for r in range(16):
                    if r >= 4:
                        sd[r - 4].wait()
                    gd[r] = pltpu.async_copy(
                        hs_hbm.at[sblk.at[r]], bufs[r % 4], gsem)
                    if r >= 1:
                        gd[r - 1].wait()
                        sd[r - 1] = pltpu.async_copy(
                            bufs[(r - 1) % 4],
                            spacc.at[dblk.at[r - 1]], ssem, add=True)
                gd[15].wait()
                sd[15] = pltpu.async_copy(
                    bufs[15 % 4], spacc.at[dblk.at[15]], ssem, add=True)
                for q in (12, 13, 14, 15):
                    sd[q].wait()

            stage(0, sblkA, dblkA)
            last = nblk - 1

            def outer(j, _):
                b0 = 2 * j
                drain(sblkA, dblkA)
                stage(jnp.minimum(b0 + 1, last), sblkB, dblkB)
                one_block(sblkA, dblkA)
                drain(sblkB, dblkB)
                stage(jnp.minimum(b0 + 2, last), sblkA, dblkA)
                one_block(sblkB, dblkB)
                return 0
            lax.fori_loop(0, nblk // 2, outer, 0)
            drain(sblkA, dblkA)
            if nblk % 2 == 1:
                one_block(sblkA, dblkA)

        for p in range(cn // 2):
            chunk = 2 * p + core
            zero_spacc()
            plsc.subcore_barrier()
            run_blocks(chunk, nblocks, sub * rows_per_tile)
            plsc.subcore_barrier()
            pltpu.sync_copy(
                spacc.at[pl.ds(sub * NPS, NPS)],
                out_hbm.at[pl.ds(sub * NPS, NPS), pl.ds(chunk * CW, CW)])

        if odd:
            chunk = cn - 1
            zero_spacc()
            plsc.subcore_barrier()
            run_blocks(chunk, nblocks // 2,
                       sub * rows_per_tile + core * half_rows)
            plsc.subcore_barrier()

            @pl.when(core == 0)
            def _w0():
                pltpu.sync_copy(
                    spacc.at[pl.ds(sub * NPS, NPS)],
                    out_hbm.at[pl.ds(sub * NPS, NPS),
                               pl.ds(chunk * CW, CW)])

            @pl.when(core == 1)
            def _w1():
                pltpu.sync_copy(spacc.at[pl.ds(sub * NPS, NPS)],
                                outx_hbm.at[pl.ds(sub * NPS, NPS), :])

    res = k(hs_rows, src2d, dst2d)
    return (res[0], res[1]) if odd else (res[0], None)


def _sc_pool(acc3, dinv, batch_pad, b3p):
    mesh = plsc.VectorSubcoreMesh(core_axis_name="c", subcore_axis_name="s")

    @functools.partial(
        pl.kernel, mesh=mesh,
        compiler_params=pltpu.CompilerParams(needs_layout_passes=False, use_tc_tiling_on_sc=False),
        out_type=_f32(NTILE, NG, F3),
        scratch_types=[
            pltpu.VMEM((NPT, 2 * LANES), jnp.float32),
            pltpu.VMEM((NPT,), jnp.float32),
            pltpu.VMEM((NPT,), jnp.int32),
            pltpu.VMEM((PCN, LANES), jnp.float32),
            pltpu.VMEM((NG + 1, 2 * LANES), jnp.float32),
        ],
    )
    def k(acc_hbm, dinv_hbm, batch_hbm, b3_hbm, out_hbm, rows, dv, bt, bb, accg):
        wid = lax.axis_index("s") * NSC + lax.axis_index("c")
        n0 = wid * NPT
        pltpu.sync_copy(dinv_hbm.at[pl.ds(n0, NPT)], dv)
        pltpu.sync_copy(batch_hbm.at[pl.ds(n0, NPT)], bt)
        pltpu.sync_copy(b3_hbm, bb)
        iota = lax.iota(jnp.int32, LANES)
        neg = jnp.full((LANES,), -jnp.inf, jnp.float32)

        iota2 = iota + LANES

        def chunk(cp, _):
            pltpu.sync_copy(
                acc_hbm.at[pl.ds(n0, NPT), pl.ds(cp * 2 * LANES, 2 * LANES)],
                rows)
            bv0 = bb[2 * cp, :]
            bv1 = bb[2 * cp + 1, :]

            def ini(i, _):
                accg[i, pl.ds(0, LANES)] = neg
                accg[i, pl.ds(LANES, LANES)] = neg
                return 0
            lax.fori_loop(0, NG + 1, ini, 0)

            def group(gidx, _):
                nb = gidx * LANES
                dvv = dv[pl.ds(nb, LANES)]
                btv = bt[pl.ds(nb, LANES)]
                for i in range(LANES):
                    r0 = rows[nb + i, pl.ds(0, LANES)]
                    r1 = rows[nb + i, pl.ds(LANES, LANES)]
                    val0 = jnp.maximum(dvv[i] * r0 + bv0, 0.0)
                    val1 = jnp.maximum(dvv[i] * r1 + bv1, 0.0)
                    gs = jnp.full((LANES,), btv[i], jnp.int32)
                    old0 = plsc.load_gather(accg, [gs, iota])
                    old1 = plsc.load_gather(accg, [gs, iota2])
                    plsc.store_scatter(accg, [gs, iota],
                                       jnp.maximum(old0, val0))
                    plsc.store_scatter(accg, [gs, iota2],
                                       jnp.maximum(old1, val1))
                return 0
            lax.fori_loop(0, NPT // LANES, group, 0)

            pltpu.sync_copy(
                accg.at[pl.ds(0, NG)],
                out_hbm.at[wid, :, pl.ds(cp * 2 * LANES, 2 * LANES)])
            return 0
        lax.fori_loop(0, PCN // 2, chunk, 0)

    return k(acc3, dinv, batch_pad, b3p.reshape(PCN, LANES))


def _sc_embed(table_pad, tgt2d):
    mesh = plsc.VectorSubcoreMesh(core_axis_name="c", subcore_axis_name="s")
    rows_per_tile = TROWS // NTILE
    nit = 25

    @functools.partial(
        pl.kernel, mesh=mesh,
        compiler_params=pltpu.CompilerParams(needs_layout_passes=False, use_tc_tiling_on_sc=False),
        out_type=_f32(1024 * 1000, ED),
        scratch_types=[
            pltpu.VMEM((10, 128), jnp.int32),
            pltpu.VMEM((128, ED), jnp.float32),
            pltpu.VMEM((128, ED), jnp.float32),
            pltpu.SemaphoreType.DMA,
        ],
    )
    def k(tab_hbm, tgt_hbm, out_hbm, idxb, gb0, gb1, sem):
        wid = lax.axis_index("s") * NSC + lax.axis_index("c")
        base = wid * rows_per_tile

        def it(i, _):
            pltpu.sync_copy(tgt_hbm.at[pl.ds(base + i * 10, 10)], idxb)
            bufs = (gb0, gb1)
            descs = [
                pltpu.async_copy(tab_hbm.at[idxb.at[0]], gb0, sem),
                pltpu.async_copy(tab_hbm.at[idxb.at[1]], gb1, sem),
            ]
            for r in range(10):
                descs[r].wait()
                pltpu.sync_copy(
                    bufs[r % 2],
                    out_hbm.at[pl.ds((base + i * 10 + r) * 128, 128)])
                if r + 2 < 10:
                    descs.append(pltpu.async_copy(
                        tab_hbm.at[idxb.at[r + 2]], bufs[r % 2], sem))
            return 0
        lax.fori_loop(0, nit, it, 0)

    return k(table_pad, tgt2d)


RB = 6272


def _tc_m1(x_pad, w1p, degp):
    def body(x_ref, w_ref, deg_ref, hs_ref, dinv_ref):
        deg = jnp.sum(deg_ref[...], axis=0)
        dinv = lax.rsqrt(jnp.maximum(deg, 0.5))
        h = jnp.dot(x_ref[...], w_ref[...], preferred_element_type=jnp.float32)
        hs_ref[...] = h * dinv[:, None]
        dinv_ref[...] = dinv[:, None]

    return pl.pallas_call(
        body,
        grid=(NP // RB,),
        in_specs=[
            pl.BlockSpec((RB, F1), lambda i: (i, 0)),
            pl.BlockSpec((F1, F1), lambda i: (0, 0)),
            pl.BlockSpec((NTILE, RB), lambda i: (0, i)),
        ],
        out_specs=[
            pl.BlockSpec((RB, F1), lambda i: (i, 0)),
            pl.BlockSpec((RB, 1), lambda i: (i, 0)),
        ],
        out_shape=[_f32(NP, F1), _f32(NP, 1)],
    )(x_pad, w1p, degp)


def _tc_m23(acc, accx, dinv, bprev, w, fin, fout):
    def body(acc_ref, accx_ref, dinv_ref, b_ref, w_ref, out_ref):
        dv = dinv_ref[...]
        full = jnp.concatenate(
            [acc_ref[:, : fin - CW], acc_ref[:, fin - CW:] + accx_ref[...]],
            axis=1)
        a = jnp.maximum(full * dv + b_ref[...][None, :], 0.0)
        h = jnp.dot(a, w_ref[...], preferred_element_type=jnp.float32)
        out_ref[...] = h * dv

    return pl.pallas_call(
        body,
        grid=(NP // RB,),
        in_specs=[
            pl.BlockSpec((RB, fin), lambda i: (i, 0)),
            pl.BlockSpec((RB, CW), lambda i: (i, 0)),
            pl.BlockSpec((RB, 1), lambda i: (i, 0)),
            pl.BlockSpec((fin,), lambda i: (0,)),
            pl.BlockSpec((fin, fout), lambda i: (0, 0)),
        ],
        out_specs=pl.BlockSpec((RB, fout), lambda i: (i, 0)),
        out_shape=_f32(NP, fout),
    )(acc, accx, dinv, bprev, w)


def _tc_pmm(emb2, u_flat):
    kb = 3200
    nk = 112000 // kb

    def body(e_ref, u_ref, out_ref):
        @pl.when(pl.program_id(0) == 0)
        def _():
            out_ref[...] = jnp.zeros_like(out_ref)
        out_ref[...] += jnp.dot(e_ref[...], u_ref[...],
                                preferred_element_type=jnp.float32)

    return pl.pallas_call(
        body,
        grid=(nk,),
        in_specs=[
            pl.BlockSpec((NG, kb), lambda k: (0, k)),
            pl.BlockSpec((kb, 128), lambda k: (k, 0)),
        ],
        out_specs=pl.BlockSpec((NG, 128), lambda k: (0, 0)),
        out_shape=_f32(NG, 128),
    )(emb2, u_flat)


def _tc_head(partials, p_raw, wg1p, bg1, wg2, bg2, bxt, wm1, bm1, wm2, bm2,
             wm3, bm3):
    bb = 128

    def body(part_ref, p_ref, wg1_ref, bg1_ref, wg2_ref, bg2_ref, bxt_ref,
             wm1_ref, bm1_ref, wm2_ref, bm2_ref, wm3_ref, bm3_ref, out_ref):
        hp = lax.Precision.HIGHEST
        pool = jnp.max(part_ref[...], axis=0)
        g = jnp.maximum(jnp.dot(pool, wg1_ref[...], precision=hp,
                                preferred_element_type=jnp.float32)
                        + bg1_ref[...][None, :], 0.0)
        g = jnp.dot(g, wg2_ref[...], precision=hp,
                    preferred_element_type=jnp.float32) \
            + bg2_ref[...][None, :]
        p = p_ref[...] + bxt_ref[...][None, :]
        hid = jnp.concatenate([g, p], axis=1)
        hid = jnp.maximum(jnp.dot(hid, wm1_ref[...], precision=hp,
                                  preferred_element_type=jnp.float32)
                          + bm1_ref[...][None, :], 0.0)
        hid = jnp.maximum(jnp.dot(hid, wm2_ref[...], precision=hp,
                                  preferred_element_type=jnp.float32)
                          + bm2_ref[...][None, :], 0.0)
        out_ref[...] = jnp.dot(hid, wm3_ref[...], precision=hp,
                               preferred_element_type=jnp.float32) \
            + bm3_ref[...][None, :]

    return pl.pallas_call(
        body,
        grid=(NG // bb,),
        in_specs=[
            pl.BlockSpec((NTILE, bb, F3), lambda i: (0, i, 0)),
            pl.BlockSpec((bb, 128), lambda i: (i, 0)),
            pl.BlockSpec((F3, 1024), lambda i: (0, 0)),
            pl.BlockSpec((1024,), lambda i: (0,)),
            pl.BlockSpec((1024, 128), lambda i: (0, 0)),
            pl.BlockSpec((128,), lambda i: (0,)),
            pl.BlockSpec((128,), lambda i: (0,)),
            pl.BlockSpec((256, 1024), lambda i: (0, 0)),
            pl.BlockSpec((1024,), lambda i: (0,)),
            pl.BlockSpec((1024, 512), lambda i: (0, 0)),
            pl.BlockSpec((512,), lambda i: (0,)),
            pl.BlockSpec((512, 1), lambda i: (0, 0)),
            pl.BlockSpec((1,), lambda i: (0,)),
        ],
        out_specs=pl.BlockSpec((bb, 1), lambda i: (i, 0)),
        out_shape=_f32(NG, 1),
    )(partials, p_raw, wg1p, bg1, wg2, bg2, bxt, wm1, bm1, wm2, bm2, wm3, bm3)


def kernel(x, edge_index, batch, target, W1, b1, W2, b2, W3, b3, Wg1, bg1,
           Wg2, bg2, emb_table, Wc, bc, Wxt, bxt, Wm1, bm1, Wm2, bm2, Wm3,
           bm3):
    f32 = jnp.float32
    x_pad = jnp.zeros((NP, F1), f32).at[:N, :78].set(x)
    w1p = jnp.zeros((F1, F1), f32).at[:78, :78].set(W1)
    w2p = jnp.zeros((F1, F2), f32).at[:78, :156].set(W2)
    w3p = jnp.zeros((F2, F3), f32).at[:156, :312].set(W3)
    b1p = jnp.zeros((F1,), f32).at[:78].set(b1)
    b2p = jnp.zeros((F2,), f32).at[:156].set(b2)
    b3p = jnp.zeros((F3,), f32).at[:312].set(b3)
    wg1p = jnp.zeros((F3, 1024), f32).at[:312, :].set(Wg1)

    loop = jnp.arange(N, dtype=jnp.int32)
    padv = jnp.full((EP - E_REAL,), N, jnp.int32)
    src = jnp.concatenate([edge_index[0], loop, padv]).reshape(EROWS, 128)
    dst = jnp.concatenate([edge_index[1], loop, padv]).reshape(EROWS, 128)
    batch_pad = jnp.concatenate(
        [batch, jnp.full((NP - N,), NG, jnp.int32)])

    table_pad = jnp.zeros((8000, ED), f32).at[:, :100].set(emb_table)
    tgt2d = target.reshape(TROWS, 128)

    wxt3 = Wxt.reshape(32, 93, 128)
    u = jnp.zeros((1000, ED, 128), f32)
    for kk in range(8):
        t_k = jnp.einsum('oc,opm->cpm', Wc[:, :, kk], wxt3,
                         precision=jax.lax.Precision.HIGHEST)
        u = u.at[:, kk:kk + 93, :].add(t_k)
    bxt_eff = bxt + jnp.einsum('o,opm->m', bc, wxt3,
                               precision=jax.lax.Precision.HIGHEST)
    u_flat = u.reshape(112000, 128)

    emb = _sc_embed(table_pad, tgt2d)
    p_raw = _tc_pmm(emb.reshape(NG, 112000), u_flat)

    degp = _sc_deg(dst).reshape(NTILE, NP)
    hs1, dinv = _tc_m1(x_pad, w1p, degp)
    acc1, acc1x = _sc_scatter(hs1.reshape(NP * CN1, CW), src, dst, CN1)
    hs2 = _tc_m23(acc1, acc1x, dinv, b1p, w2p, F1, F2)
    acc2, acc2x = _sc_scatter(hs2.reshape(NP * CN2, CW), src, dst, CN2)
    hs3 = _tc_m23(acc2, acc2x, dinv, b2p, w3p, F2, F3)
    acc3, _unused = _sc_scatter(hs3.reshape(NP * CN3, CW), src, dst, CN3)
    partials = _sc_pool(acc3, dinv.reshape(NP), batch_pad, b3p)

    return _tc_head(partials, p_raw, wg1p, bg1, Wg2, bg2, bxt_eff, Wm1, bm1,
                    Wm2, bm2, Wm3, bm3)
